# baseline (speedup 1.0000x reference)
import functools

import jax
import jax.numpy as jnp
from jax import lax
from jax.experimental import pallas as pl
from jax.experimental.pallas import tpu as pltpu
from jax.experimental.pallas import tpu_sc as plsc

_N = 10000
_E = 160000
_H = 128
_NH = 10
_NF = 32
_D = _NH * _NF
_NHP = 16

_DP = 384
_GP = 512
_GI = _GP // 2
_RP = 128

_NC = 2
_NS = 16
_NW = _NC * _NS
_CH = 128


def _ssp(x):
    return jnp.maximum(x, 0.0) + jnp.log1p(jnp.exp(-jnp.abs(x))) - jnp.log(2.0).astype(x.dtype)


def _pack_bf16(x):
    k = x.shape[1] // 2
    lo = jax.lax.bitcast_convert_type(x[:, :k].astype(jnp.bfloat16), jnp.uint16)
    hi = jax.lax.bitcast_convert_type(x[:, k:].astype(jnp.bfloat16), jnp.uint16)
    return (hi.astype(jnp.int32) << 16) | lo.astype(jnp.int32)


def _unpack_bf16(x):
    lo = jax.lax.bitcast_convert_type((x & 0xFFFF).astype(jnp.uint16),
                                      jnp.bfloat16)
    hi = jax.lax.bitcast_convert_type(
        jax.lax.shift_right_logical(x, 16).astype(jnp.uint16), jnp.bfloat16)
    return jnp.concatenate([lo.astype(jnp.float32), hi.astype(jnp.float32)],
                           axis=1)


def _tc1_body(v_ref, wt_ref, ar_ref, vv_ref, rt_ref):
    vv = jnp.dot(v_ref[...], wt_ref[...], preferred_element_type=jnp.float32)
    vv_ref[...] = _pack_bf16(vv)
    rt_ref[...] = jnp.dot(vv[:, :_DP], ar_ref[...],
                          preferred_element_type=jnp.float32)


def _tc2_body(det_ref, gv_ref, gr_ref, w0t_ref, b0_ref, w2t_ref, b2_ref,
              ae_ref, al_ref, sx_ref, out_ref):
    h = jax.lax.dot_general(det_ref[...], w0t_ref[...],
                            (((0,), (0,)), ((), ())),
                            preferred_element_type=jnp.float32)
    h = _ssp(h + b0_ref[...])
    w = jnp.dot(h, w2t_ref[...], preferred_element_type=jnp.float32) + b2_ref[...]
    gv = _unpack_bf16(gv_ref[...])[:, :_DP]
    ee = jnp.dot(w, ae_ref[...], preferred_element_type=jnp.float32)
    el = jnp.dot(gv, al_ref[...], preferred_element_type=jnp.float32)
    s = _ssp(el + ee + gr_ref[...])
    sx = jnp.dot(s, sx_ref[...], preferred_element_type=jnp.float32)
    out_ref[...] = jnp.transpose(gv * sx * w)[:_D, :]


def _sc_gather(vv, rt, idxj, idxi, ep, nb, n0):
    mesh = plsc.VectorSubcoreMesh(core_axis_name="c", subcore_axis_name="s")
    n1 = nb - n0
    nmax = max(n0, n1)

    @functools.partial(
        pl.kernel,
        out_type=[
            jax.ShapeDtypeStruct((ep, _GI), jnp.int32),
            jax.ShapeDtypeStruct((ep, _RP), jnp.float32),
        ],
        mesh=mesh,
        scratch_types=[
            pltpu.VMEM((nmax, _CH), jnp.int32),
            pltpu.VMEM((nmax, _CH), jnp.int32),
            pltpu.VMEM((_CH, _GI), jnp.int32),
            pltpu.VMEM((_CH, _GI), jnp.int32),
            pltpu.VMEM((_CH, _RP), jnp.float32),
            pltpu.SemaphoreType.DMA,
            pltpu.SemaphoreType.DMA,
            pltpu.SemaphoreType.DMA,
            pltpu.SemaphoreType.DMA,
            pltpu.SemaphoreType.DMA,
        ],
    )
    def k(vv_hbm, rt_hbm, idxj_hbm, idxi_hbm, gv_hbm, gr_hbm,
          idxj_v, idxi_v, rv0, rv1, rrow_v, gs0, gs1, go0, go1, semr):
        cid = lax.axis_index("c")
        sid = lax.axis_index("s")
        crow = sid * nb + jnp.where(cid == 0, 0, n0)
        my_n = jnp.where(cid == 0, n0, n1)
        ng = my_n // 2

        @pl.when(cid == 0)
        def _():
            pltpu.sync_copy(idxj_hbm.at[pl.ds(crow, n0)], idxj_v.at[pl.ds(0, n0)])
            pltpu.sync_copy(idxi_hbm.at[pl.ds(crow, n0)], idxi_v.at[pl.ds(0, n0)])

        @pl.when(cid == 1)
        def _():
            pltpu.sync_copy(idxj_hbm.at[pl.ds(crow, n1)], idxj_v.at[pl.ds(0, n1)])
            pltpu.sync_copy(idxi_hbm.at[pl.ds(crow, n1)], idxi_v.at[pl.ds(0, n1)])

        def g_rows(kk, buf, sem):
            return pltpu.make_async_copy(vv_hbm.at[idxj_v.at[kk]], buf, sem)

        def o_rows(kk, buf, sem):
            return pltpu.make_async_copy(
                buf, gv_hbm.at[pl.ds((crow + kk) * _CH, _CH)], sem)

        def small(kk):
            pltpu.async_copy(rt_hbm.at[idxi_v.at[kk]], rrow_v, semr).wait()
            pltpu.sync_copy(rrow_v, gr_hbm.at[pl.ds((crow + kk) * _CH, _CH)])

        g_rows(0, rv0, gs0).start()

        def body(g, carry):
            k0 = 2 * g
            k1 = k0 + 1
            g_rows(k0, rv0, gs0).wait()
            o_rows(k0, rv0, go0).start()
            small(k0)

            @pl.when(g > 0)
            def _():
                o_rows(k1 - 2, rv1, go1).wait()

            g_rows(k1, rv1, gs1).start()
            g_rows(k1, rv1, gs1).wait()
            o_rows(k1, rv1, go1).start()
            small(k1)
            o_rows(k0, rv0, go0).wait()

            @pl.when(g < ng - 1)
            def _():
                g_rows(k0 + 2, rv0, gs0).start()

            return carry

        lax.fori_loop(0, ng, body, 0)
        o_rows(my_n - 1, rv1, go1).wait()

    return k(vv, rt, idxj, idxi)


def _blockdiag(a, nrow=_D, ncol=_NHP):
    out = jnp.zeros((nrow, ncol), jnp.float32)
    rows = jnp.arange(_D)
    cols = jnp.repeat(jnp.arange(_NH), _NF)
    return out.at[rows, cols].set(a.reshape(-1))


def kernel(v, dist, dist_emb, edge_index, lin_w, attn_l, attn_r, attn_edge,
           mlp_w0, mlp_b0, mlp_w2, mlp_b2):
    n, h = v.shape
    e = dist_emb.shape[0]

    a_l = _blockdiag(attn_l, _DP, _RP)
    a_r = _blockdiag(attn_r, _DP, _RP)
    a_e = _blockdiag(attn_edge, _DP, _RP)
    s_exp = jnp.zeros((_RP, _DP), jnp.float32)
    s_exp = s_exp.at[jnp.repeat(jnp.arange(_NH), _NF), jnp.arange(_D)].set(1.0)
    w_t = jnp.pad(lin_w.T, ((0, 0), (0, _GP - _D)))
    w0t = mlp_w0.T
    w2t = jnp.pad(mlp_w2.T, ((0, 0), (0, _DP - _D)))
    b0 = mlp_b0.reshape(1, _NF)
    b2 = jnp.pad(mlp_b2.reshape(1, _D), ((0, 0), (0, _DP - _D)))

    nb = 2 * -(-e // (_NS * 2 * _CH))
    ep = nb * _NS * _CH
    n0 = nb // 2 // 8 * 8
    j_idx = jnp.pad(edge_index[0], (0, ep - e)).reshape(ep // _CH, _CH)
    i_idx = jnp.pad(edge_index[1], (0, ep - e)).reshape(ep // _CH, _CH)

    vv, rt = pl.pallas_call(
        _tc1_body,
        out_shape=[
            jax.ShapeDtypeStruct((n, _GI), jnp.int32),
            jax.ShapeDtypeStruct((n, _RP), jnp.float32),
        ],
    )(v, w_t, a_r)

    gv, gr = _sc_gather(vv, rt, j_idx, i_idx, ep, nb, n0)

    be = 3200
    grid = e // be
    full = lambda shp: pl.BlockSpec(shp, lambda g: (0, 0))
    out_t = pl.pallas_call(
        _tc2_body,
        grid=(grid,),
        in_specs=[
            pl.BlockSpec((50, be), lambda g: (0, g)),
            pl.BlockSpec((be, _GI), lambda g: (g, 0)),
            pl.BlockSpec((be, _RP), lambda g: (g, 0)),
            full((50, _NF)),
            full((1, _NF)),
            full((_NF, _DP)),
            full((1, _DP)),
            full((_DP, _RP)),
            full((_DP, _RP)),
            full((_RP, _DP)),
        ],
        out_specs=pl.BlockSpec((_D, be), lambda g: (0, g)),
        out_shape=jax.ShapeDtypeStruct((_D, e), jnp.float32),
    )(dist_emb.T, gv, gr, w0t, b0, w2t, b2, a_e, a_l, s_exp)

    return jnp.transpose(out_t.reshape(_NH, _NF, e), (2, 0, 1))


# speedup vs baseline: 1.4680x; 1.4680x over previous
import functools

import jax
import jax.numpy as jnp
from jax import lax
from jax.experimental import pallas as pl
from jax.experimental.pallas import tpu as pltpu
from jax.experimental.pallas import tpu_sc as plsc

_N = 10000
_E = 160000
_H = 128
_NH = 10
_NF = 32
_D = _NH * _NF
_NHP = 16

_DP = 384
_GP = 512
_GI = _GP // 2
_RP = 128

_NC = 2
_NS = 16
_NW = _NC * _NS
_CH = 64


def _ssp(x):
    return jnp.maximum(x, 0.0) + jnp.log1p(jnp.exp(-jnp.abs(x))) - jnp.log(2.0).astype(x.dtype)


def _pack_bf16(x):
    k = x.shape[1] // 2
    lo = jax.lax.bitcast_convert_type(x[:, :k].astype(jnp.bfloat16), jnp.uint16)
    hi = jax.lax.bitcast_convert_type(x[:, k:].astype(jnp.bfloat16), jnp.uint16)
    return (hi.astype(jnp.int32) << 16) | lo.astype(jnp.int32)


def _unpack_bf16(x):
    lo = jax.lax.bitcast_convert_type((x & 0xFFFF).astype(jnp.uint16),
                                      jnp.bfloat16)
    hi = jax.lax.bitcast_convert_type(
        jax.lax.shift_right_logical(x, 16).astype(jnp.uint16), jnp.bfloat16)
    return jnp.concatenate([lo.astype(jnp.float32), hi.astype(jnp.float32)],
                           axis=1)


def _tc1_body(v_ref, wt_ref, ar_ref, vv_ref, rt_ref):
    vv = jnp.dot(v_ref[...], wt_ref[...], preferred_element_type=jnp.float32)
    vv_ref[...] = _pack_bf16(vv)
    rt_ref[...] = jnp.dot(vv[:, :_DP], ar_ref[...],
                          preferred_element_type=jnp.float32)


def _tc2_body(det_ref, gv_ref, gr_ref, w0t_ref, b0_ref, w2t_ref, b2_ref,
              ae_ref, al_ref, sx_ref, out_ref):
    h = jax.lax.dot_general(det_ref[...], w0t_ref[...],
                            (((0,), (0,)), ((), ())),
                            preferred_element_type=jnp.float32)
    h = _ssp(h + b0_ref[...])
    w = jnp.dot(h, w2t_ref[...], preferred_element_type=jnp.float32) + b2_ref[...]
    gv = _unpack_bf16(gv_ref[...])[:, :_DP]
    ee = jnp.dot(w, ae_ref[...], preferred_element_type=jnp.float32)
    el = jnp.dot(gv, al_ref[...], preferred_element_type=jnp.float32)
    s = _ssp(el + ee + gr_ref[...])
    sx = jnp.dot(s, sx_ref[...], preferred_element_type=jnp.float32)
    out_ref[...] = jnp.transpose(gv * sx * w)[:_D, :]


def _sc_gather(vv, rt, idxj, idxi, ep, nb, n0):
    mesh = plsc.VectorSubcoreMesh(core_axis_name="c", subcore_axis_name="s")
    n1 = nb - n0
    nmax = max(n0, n1)
    nbuf = 4

    @functools.partial(
        pl.kernel,
        out_type=[
            jax.ShapeDtypeStruct((ep, _GI), jnp.int32),
            jax.ShapeDtypeStruct((ep, _RP), jnp.float32),
        ],
        mesh=mesh,
        scratch_types=[
            pltpu.VMEM((nmax, _CH), jnp.int32),
            pltpu.VMEM((nmax, _CH), jnp.int32),
            [pltpu.VMEM((_CH, _GI), jnp.int32)] * nbuf,
            [pltpu.VMEM((_CH, _RP), jnp.float32)] * nbuf,
            [pltpu.SemaphoreType.DMA] * nbuf,
            [pltpu.SemaphoreType.DMA] * nbuf,
            [pltpu.SemaphoreType.DMA] * nbuf,
            [pltpu.SemaphoreType.DMA] * nbuf,
        ],
    )
    def k(vv_hbm, rt_hbm, idxj_hbm, idxi_hbm, gv_hbm, gr_hbm,
          idxj_v, idxi_v, rv, rr, gs, go, hs, ho):
        cid = lax.axis_index("c")
        sid = lax.axis_index("s")
        crow = sid * nb + jnp.where(cid == 0, 0, n0)
        my_n = jnp.where(cid == 0, n0, n1)

        @pl.when(cid == 0)
        def _():
            pltpu.sync_copy(idxj_hbm.at[pl.ds(crow, n0)], idxj_v.at[pl.ds(0, n0)])
            pltpu.sync_copy(idxi_hbm.at[pl.ds(crow, n0)], idxi_v.at[pl.ds(0, n0)])

        @pl.when(cid == 1)
        def _():
            pltpu.sync_copy(idxj_hbm.at[pl.ds(crow, n1)], idxj_v.at[pl.ds(0, n1)])
            pltpu.sync_copy(idxi_hbm.at[pl.ds(crow, n1)], idxi_v.at[pl.ds(0, n1)])

        def g_rows(kk, b):
            return pltpu.make_async_copy(vv_hbm.at[idxj_v.at[kk]], rv[b], gs[b])

        def o_rows(kk, b):
            return pltpu.make_async_copy(
                rv[b], gv_hbm.at[pl.ds((crow + kk) * _CH, _CH)], go[b])

        def g_small(kk, b):
            return pltpu.make_async_copy(rt_hbm.at[idxi_v.at[kk]], rr[b], hs[b])

        def o_small(kk, b):
            return pltpu.make_async_copy(
                rr[b], gr_hbm.at[pl.ds((crow + kk) * _CH, _CH)], ho[b])

        for b in range(nbuf):
            g_rows(b, b).start()
            g_small(b, b).start()

        def body(g, carry):
            for b in range(nbuf):
                kk = nbuf * g + b
                g_rows(kk, b).wait()
                o_rows(kk, b).start()
                g_small(kk, b).wait()
                o_small(kk, b).start()

                @pl.when(kk + nbuf < my_n)
                def _():
                    o_rows(kk, b).wait()
                    g_rows(kk + nbuf, b).start()
                    o_small(kk, b).wait()
                    g_small(kk + nbuf, b).start()

            return carry

        lax.fori_loop(0, my_n // nbuf, body, 0)
        for b in range(nbuf):
            @pl.when(my_n - nbuf + b >= 0)
            def _(b=b):
                o_rows(my_n - nbuf + b, b).wait()
                o_small(my_n - nbuf + b, b).wait()

    return k(vv, rt, idxj, idxi)


def _blockdiag(a, nrow=_D, ncol=_NHP):
    out = jnp.zeros((nrow, ncol), jnp.float32)
    rows = jnp.arange(_D)
    cols = jnp.repeat(jnp.arange(_NH), _NF)
    return out.at[rows, cols].set(a.reshape(-1))


def kernel(v, dist, dist_emb, edge_index, lin_w, attn_l, attn_r, attn_edge,
           mlp_w0, mlp_b0, mlp_w2, mlp_b2):
    n, h = v.shape
    e = dist_emb.shape[0]

    a_l = _blockdiag(attn_l, _DP, _RP)
    a_r = _blockdiag(attn_r, _DP, _RP)
    a_e = _blockdiag(attn_edge, _DP, _RP)
    s_exp = jnp.zeros((_RP, _DP), jnp.float32)
    s_exp = s_exp.at[jnp.repeat(jnp.arange(_NH), _NF), jnp.arange(_D)].set(1.0)
    w_t = jnp.pad(lin_w.T, ((0, 0), (0, _GP - _D)))
    w0t = mlp_w0.T
    w2t = jnp.pad(mlp_w2.T, ((0, 0), (0, _DP - _D)))
    b0 = mlp_b0.reshape(1, _NF)
    b2 = jnp.pad(mlp_b2.reshape(1, _D), ((0, 0), (0, _DP - _D)))

    nb = 16 * -(-e // (_NS * 16 * _CH))
    ep = nb * _NS * _CH
    n0 = nb // 2 // 8 * 8
    j_idx = jnp.pad(edge_index[0], (0, ep - e)).reshape(ep // _CH, _CH)
    i_idx = jnp.pad(edge_index[1], (0, ep - e)).reshape(ep // _CH, _CH)

    vv, rt = pl.pallas_call(
        _tc1_body,
        out_shape=[
            jax.ShapeDtypeStruct((n, _GI), jnp.int32),
            jax.ShapeDtypeStruct((n, _RP), jnp.float32),
        ],
    )(v, w_t, a_r)

    gv, gr = _sc_gather(vv, rt, j_idx, i_idx, ep, nb, n0)

    be = 3200
    grid = e // be
    full = lambda shp: pl.BlockSpec(shp, lambda g: (0, 0))
    out_t = pl.pallas_call(
        _tc2_body,
        grid=(grid,),
        in_specs=[
            pl.BlockSpec((50, be), lambda g: (0, g)),
            pl.BlockSpec((be, _GI), lambda g: (g, 0)),
            pl.BlockSpec((be, _RP), lambda g: (g, 0)),
            full((50, _NF)),
            full((1, _NF)),
            full((_NF, _DP)),
            full((1, _DP)),
            full((_DP, _RP)),
            full((_DP, _RP)),
            full((_RP, _DP)),
        ],
        out_specs=pl.BlockSpec((_D, be), lambda g: (0, g)),
        out_shape=jax.ShapeDtypeStruct((_D, e), jnp.float32),
    )(dist_emb.T, gv, gr, w0t, b0, w2t, b2, a_e, a_l, s_exp)

    return jnp.transpose(out_t.reshape(_NH, _NF, e), (2, 0, 1))


# speedup vs baseline: 1.4762x; 1.0056x over previous
import functools

import jax
import jax.numpy as jnp
from jax import lax
from jax.experimental import pallas as pl
from jax.experimental.pallas import tpu as pltpu
from jax.experimental.pallas import tpu_sc as plsc

_N = 10000
_E = 160000
_H = 128
_NH = 10
_NF = 32
_D = _NH * _NF
_NHP = 16

_DP = 384
_GP = 512
_GI = _GP // 2
_RP = 128

_NC = 2
_NS = 16
_NW = _NC * _NS
_CH = 64


def _ssp(x):
    return jnp.maximum(x, 0.0) + jnp.log1p(jnp.exp(-jnp.abs(x))) - jnp.log(2.0).astype(x.dtype)


def _pack_bf16(x):
    k = x.shape[1] // 2
    lo = jax.lax.bitcast_convert_type(x[:, :k].astype(jnp.bfloat16), jnp.uint16)
    hi = jax.lax.bitcast_convert_type(x[:, k:].astype(jnp.bfloat16), jnp.uint16)
    return (hi.astype(jnp.int32) << 16) | lo.astype(jnp.int32)


def _unpack_bf16(x):
    lo = jax.lax.bitcast_convert_type((x & 0xFFFF).astype(jnp.uint16),
                                      jnp.bfloat16)
    hi = jax.lax.bitcast_convert_type(
        jax.lax.shift_right_logical(x, 16).astype(jnp.uint16), jnp.bfloat16)
    return jnp.concatenate([lo.astype(jnp.float32), hi.astype(jnp.float32)],
                           axis=1)


def _tc1_body(v_ref, wt_ref, ar_ref, vv_ref, rt_ref):
    vv = jnp.dot(v_ref[...], wt_ref[...], preferred_element_type=jnp.float32)
    vv_ref[...] = _pack_bf16(vv)
    rt_ref[...] = jnp.dot(vv[:, :_DP], ar_ref[...],
                          preferred_element_type=jnp.float32)


def _tc2_body(det_ref, gv_ref, gr_ref, w0t_ref, b0_ref, w2t_ref, b2_ref,
              ae_ref, al_ref, sx_ref, out_ref):
    h = jax.lax.dot_general(det_ref[...], w0t_ref[...],
                            (((0,), (0,)), ((), ())),
                            preferred_element_type=jnp.float32)
    h = _ssp(h + b0_ref[...])
    w = jnp.dot(h, w2t_ref[...], preferred_element_type=jnp.float32) + b2_ref[...]
    gv = _unpack_bf16(gv_ref[...])[:, :_DP]
    ee = jnp.dot(w, ae_ref[...], preferred_element_type=jnp.float32)
    el = jnp.dot(gv, al_ref[...], preferred_element_type=jnp.float32)
    s = _ssp(el + ee + gr_ref[:, :_NHP])
    sx = jnp.dot(s, sx_ref[...], preferred_element_type=jnp.float32)
    out_ref[...] = jnp.transpose(gv * sx * w)[:_D, :]


def _sc_gather(vv, rt, idxj, idxi, ep, nb, n0):
    mesh = plsc.VectorSubcoreMesh(core_axis_name="c", subcore_axis_name="s")
    n1 = nb - n0
    nmax = max(n0, n1)
    nbuf = 4

    @functools.partial(
        pl.kernel,
        out_type=[
            jax.ShapeDtypeStruct((ep, _GI), jnp.int32),
            jax.ShapeDtypeStruct((ep, _RP), jnp.float32),
        ],
        mesh=mesh,
        scratch_types=[
            pltpu.VMEM((nmax, _CH), jnp.int32),
            pltpu.VMEM((nmax, _CH), jnp.int32),
            [pltpu.VMEM((_CH, _GI), jnp.int32)] * nbuf,
            [pltpu.VMEM((_CH, _RP), jnp.float32)] * nbuf,
            [pltpu.SemaphoreType.DMA] * nbuf,
            [pltpu.SemaphoreType.DMA] * nbuf,
            [pltpu.SemaphoreType.DMA] * nbuf,
            [pltpu.SemaphoreType.DMA] * nbuf,
        ],
    )
    def k(vv_hbm, rt_hbm, idxj_hbm, idxi_hbm, gv_hbm, gr_hbm,
          idxj_v, idxi_v, rv, rr, gs, go, hs, ho):
        cid = lax.axis_index("c")
        sid = lax.axis_index("s")
        crow = sid * nb + jnp.where(cid == 0, 0, n0)
        my_n = jnp.where(cid == 0, n0, n1)

        @pl.when(cid == 0)
        def _():
            pltpu.sync_copy(idxj_hbm.at[pl.ds(crow, n0)], idxj_v.at[pl.ds(0, n0)])
            pltpu.sync_copy(idxi_hbm.at[pl.ds(crow, n0)], idxi_v.at[pl.ds(0, n0)])

        @pl.when(cid == 1)
        def _():
            pltpu.sync_copy(idxj_hbm.at[pl.ds(crow, n1)], idxj_v.at[pl.ds(0, n1)])
            pltpu.sync_copy(idxi_hbm.at[pl.ds(crow, n1)], idxi_v.at[pl.ds(0, n1)])

        def g_rows(kk, b):
            return pltpu.make_async_copy(vv_hbm.at[idxj_v.at[kk]], rv[b], gs[b])

        def o_rows(kk, b):
            return pltpu.make_async_copy(
                rv[b], gv_hbm.at[pl.ds((crow + kk) * _CH, _CH)], go[b])

        def g_small(kk, b):
            return pltpu.make_async_copy(rt_hbm.at[idxi_v.at[kk]], rr[b], hs[b])

        def o_small(kk, b):
            return pltpu.make_async_copy(
                rr[b], gr_hbm.at[pl.ds((crow + kk) * _CH, _CH)], ho[b])

        for b in range(nbuf):
            g_rows(b, b).start()
            g_small(b, b).start()

        def body(g, carry):
            for b in range(nbuf):
                kk = nbuf * g + b
                g_rows(kk, b).wait()
                o_rows(kk, b).start()
                g_small(kk, b).wait()
                o_small(kk, b).start()

                @pl.when(kk + nbuf < my_n)
                def _():
                    o_rows(kk, b).wait()
                    g_rows(kk + nbuf, b).start()
                    o_small(kk, b).wait()
                    g_small(kk + nbuf, b).start()

            return carry

        lax.fori_loop(0, my_n // nbuf, body, 0)
        for b in range(nbuf):
            @pl.when(my_n - nbuf + b >= 0)
            def _(b=b):
                o_rows(my_n - nbuf + b, b).wait()
                o_small(my_n - nbuf + b, b).wait()

    return k(vv, rt, idxj, idxi)


def _blockdiag(a, nrow=_D, ncol=_NHP):
    out = jnp.zeros((nrow, ncol), jnp.float32)
    rows = jnp.arange(_D)
    cols = jnp.repeat(jnp.arange(_NH), _NF)
    return out.at[rows, cols].set(a.reshape(-1))


def kernel(v, dist, dist_emb, edge_index, lin_w, attn_l, attn_r, attn_edge,
           mlp_w0, mlp_b0, mlp_w2, mlp_b2):
    n, h = v.shape
    e = dist_emb.shape[0]

    a_l = _blockdiag(attn_l, _DP, _NHP)
    a_r = _blockdiag(attn_r, _DP, _RP)
    a_e = _blockdiag(attn_edge, _DP, _NHP)
    s_exp = jnp.zeros((_NHP, _DP), jnp.float32)
    s_exp = s_exp.at[jnp.repeat(jnp.arange(_NH), _NF), jnp.arange(_D)].set(1.0)
    w_t = jnp.pad(lin_w.T, ((0, 0), (0, _GP - _D)))
    w0t = mlp_w0.T
    w2t = jnp.pad(mlp_w2.T, ((0, 0), (0, _DP - _D)))
    b0 = mlp_b0.reshape(1, _NF)
    b2 = jnp.pad(mlp_b2.reshape(1, _D), ((0, 0), (0, _DP - _D)))

    nb = 16 * -(-e // (_NS * 16 * _CH))
    ep = nb * _NS * _CH
    n0 = (nb * 13) // 20 // 8 * 8
    j_idx = jnp.pad(edge_index[0], (0, ep - e)).reshape(ep // _CH, _CH)
    i_idx = jnp.pad(edge_index[1], (0, ep - e)).reshape(ep // _CH, _CH)

    vv, rt = pl.pallas_call(
        _tc1_body,
        out_shape=[
            jax.ShapeDtypeStruct((n, _GI), jnp.int32),
            jax.ShapeDtypeStruct((n, _RP), jnp.float32),
        ],
    )(v, w_t, a_r)

    gv, gr = _sc_gather(vv, rt, j_idx, i_idx, ep, nb, n0)

    be = 3200
    grid = e // be
    full = lambda shp: pl.BlockSpec(shp, lambda g: (0, 0))
    out_t = pl.pallas_call(
        _tc2_body,
        grid=(grid,),
        in_specs=[
            pl.BlockSpec((50, be), lambda g: (0, g)),
            pl.BlockSpec((be, _GI), lambda g: (g, 0)),
            pl.BlockSpec((be, _RP), lambda g: (g, 0)),
            full((50, _NF)),
            full((1, _NF)),
            full((_NF, _DP)),
            full((1, _DP)),
            full((_DP, _NHP)),
            full((_DP, _NHP)),
            full((_NHP, _DP)),
        ],
        out_specs=pl.BlockSpec((_D, be), lambda g: (0, g)),
        out_shape=jax.ShapeDtypeStruct((_D, e), jnp.float32),
    )(dist_emb.T, gv, gr, w0t, b0, w2t, b2, a_e, a_l, s_exp)

    return jnp.transpose(out_t.reshape(_NH, _NF, e), (2, 0, 1))


# speedup vs baseline: 1.5930x; 1.0791x over previous
import functools

import jax
import jax.numpy as jnp
from jax import lax
from jax.experimental import pallas as pl
from jax.experimental.pallas import tpu as pltpu
from jax.experimental.pallas import tpu_sc as plsc

_N = 10000
_E = 160000
_H = 128
_NH = 10
_NF = 32
_D = _NH * _NF
_NHP = 16

_DP = 384
_GP = 512
_GI = _GP // 2
_RP = 128

_NC = 2
_NS = 16
_NW = _NC * _NS
_CH = 64


def _ssp(x):
    return jnp.maximum(x, 0.0) + jnp.log1p(jnp.exp(-jnp.abs(x))) - jnp.log(2.0).astype(x.dtype)


def _pack_bf16(x):
    k = x.shape[1] // 2
    lo = jax.lax.bitcast_convert_type(x[:, :k].astype(jnp.bfloat16), jnp.uint16)
    hi = jax.lax.bitcast_convert_type(x[:, k:].astype(jnp.bfloat16), jnp.uint16)
    return (hi.astype(jnp.int32) << 16) | lo.astype(jnp.int32)


def _unpack_bf16(x):
    lo = jax.lax.bitcast_convert_type((x & 0xFFFF).astype(jnp.uint16),
                                      jnp.bfloat16)
    hi = jax.lax.bitcast_convert_type(
        jax.lax.shift_right_logical(x, 16).astype(jnp.uint16), jnp.bfloat16)
    return jnp.concatenate([lo.astype(jnp.float32), hi.astype(jnp.float32)],
                           axis=1)


def _tc1_body(v_ref, wt_ref, ar_ref, vv_ref, rt_ref):
    vv = jnp.dot(v_ref[...], wt_ref[...], preferred_element_type=jnp.float32)
    vv_ref[...] = _pack_bf16(vv)
    rt_ref[...] = jnp.dot(vv[:, :_DP], ar_ref[...],
                          preferred_element_type=jnp.float32)


def _tc2_body(det_ref, gv_ref, gr_ref, w0t_ref, b0_ref, w2t_ref, b2_ref,
              ae_ref, al_ref, sx_ref, out_ref):
    h = jax.lax.dot_general(det_ref[...], w0t_ref[...],
                            (((0,), (0,)), ((), ())),
                            preferred_element_type=jnp.float32)
    h = _ssp(h + b0_ref[...])
    w = jnp.dot(h, w2t_ref[...], preferred_element_type=jnp.float32) + b2_ref[...]
    gv = _unpack_bf16(gv_ref[...])[:, :_DP]
    ee = jnp.dot(w, ae_ref[...], preferred_element_type=jnp.float32)
    el = jnp.dot(gv, al_ref[...], preferred_element_type=jnp.float32)
    s = _ssp(el + ee + gr_ref[:, :_NHP])
    sx = jnp.dot(s, sx_ref[...], preferred_element_type=jnp.float32)
    out_ref[...] = jnp.transpose(gv * sx * w)[:_D, :]


def _sc_gather(vv, rt, idxj, idxi, ep, nb, n0):
    mesh = plsc.VectorSubcoreMesh(core_axis_name="c", subcore_axis_name="s")
    n1 = nb - n0
    nmax = max(n0, n1)
    nbuf = 4

    @functools.partial(
        pl.kernel,
        out_type=[
            jax.ShapeDtypeStruct((ep, _GI), jnp.int32),
            jax.ShapeDtypeStruct((ep, _RP), jnp.float32),
        ],
        mesh=mesh,
        scratch_types=[
            pltpu.VMEM((nmax, _CH), jnp.int32),
            pltpu.VMEM((nmax, _CH), jnp.int32),
            [pltpu.VMEM((_CH, _GI), jnp.int32)] * nbuf,
            [pltpu.VMEM((_CH, _RP), jnp.float32)] * nbuf,
            [pltpu.SemaphoreType.DMA] * nbuf,
            [pltpu.SemaphoreType.DMA] * nbuf,
            [pltpu.SemaphoreType.DMA] * nbuf,
            [pltpu.SemaphoreType.DMA] * nbuf,
        ],
    )
    def k(vv_hbm, rt_hbm, idxj_hbm, idxi_hbm, gv_hbm, gr_hbm,
          idxj_v, idxi_v, rv, rr, gs, go, hs, ho):
        cid = lax.axis_index("c")
        sid = lax.axis_index("s")
        crow = sid * nb + jnp.where(cid == 0, 0, n0)
        my_n = jnp.where(cid == 0, n0, n1)

        @pl.when(cid == 0)
        def _():
            pltpu.sync_copy(idxj_hbm.at[pl.ds(crow, n0)], idxj_v.at[pl.ds(0, n0)])
            pltpu.sync_copy(idxi_hbm.at[pl.ds(crow, n0)], idxi_v.at[pl.ds(0, n0)])

        @pl.when(cid == 1)
        def _():
            pltpu.sync_copy(idxj_hbm.at[pl.ds(crow, n1)], idxj_v.at[pl.ds(0, n1)])
            pltpu.sync_copy(idxi_hbm.at[pl.ds(crow, n1)], idxi_v.at[pl.ds(0, n1)])

        def g_rows(kk, b):
            return pltpu.make_async_copy(vv_hbm.at[idxj_v.at[kk]], rv[b], gs[b])

        def o_rows(kk, b):
            return pltpu.make_async_copy(
                rv[b], gv_hbm.at[pl.ds((crow + kk) * _CH, _CH)], go[b])

        def g_small(kk, b):
            return pltpu.make_async_copy(rt_hbm.at[idxi_v.at[kk]], rr[b], hs[b])

        def o_small(kk, b):
            return pltpu.make_async_copy(
                rr[b], gr_hbm.at[pl.ds((crow + kk) * _CH, _CH)], ho[b])

        for b in range(nbuf):
            g_rows(b, b).start()
            g_small(b, b).start()

        def body(g, carry):
            for b in range(nbuf):
                kk = nbuf * g + b
                g_rows(kk, b).wait()
                o_rows(kk, b).start()
                g_small(kk, b).wait()
                o_small(kk, b).start()

                @pl.when(kk + nbuf < my_n)
                def _():
                    o_rows(kk, b).wait()
                    g_rows(kk + nbuf, b).start()
                    o_small(kk, b).wait()
                    g_small(kk + nbuf, b).start()

            return carry

        lax.fori_loop(0, my_n // nbuf, body, 0)
        for b in range(nbuf):
            @pl.when(my_n - nbuf + b >= 0)
            def _(b=b):
                o_rows(my_n - nbuf + b, b).wait()
                o_small(my_n - nbuf + b, b).wait()

    return k(vv, rt, idxj, idxi)


def _blockdiag(a, nrow=_D, ncol=_NHP):
    out = jnp.zeros((nrow, ncol), jnp.float32)
    rows = jnp.arange(_D)
    cols = jnp.repeat(jnp.arange(_NH), _NF)
    return out.at[rows, cols].set(a.reshape(-1))


def kernel(v, dist, dist_emb, edge_index, lin_w, attn_l, attn_r, attn_edge,
           mlp_w0, mlp_b0, mlp_w2, mlp_b2):
    n, h = v.shape
    e = dist_emb.shape[0]

    a_l = _blockdiag(attn_l, _DP, _NHP)
    a_r = _blockdiag(attn_r, _DP, _RP)
    a_e = _blockdiag(attn_edge, _DP, _NHP)
    s_exp = jnp.zeros((_NHP, _DP), jnp.float32)
    s_exp = s_exp.at[jnp.repeat(jnp.arange(_NH), _NF), jnp.arange(_D)].set(1.0)
    w_t = jnp.pad(lin_w.T, ((0, 0), (0, _GP - _D)))
    w0t = mlp_w0.T
    w2t = jnp.pad(mlp_w2.T, ((0, 0), (0, _DP - _D)))
    b0 = mlp_b0.reshape(1, _NF)
    b2 = jnp.pad(mlp_b2.reshape(1, _D), ((0, 0), (0, _DP - _D)))

    nb = 16 * -(-e // (_NS * 16 * _CH))
    ep = nb * _NS * _CH
    n0 = (nb * 13) // 20 // 8 * 8
    j_idx = jnp.pad(edge_index[0], (0, ep - e)).reshape(ep // _CH, _CH)
    i_idx = jnp.pad(edge_index[1], (0, ep - e)).reshape(ep // _CH, _CH)

    vv, rt = pl.pallas_call(
        _tc1_body,
        out_shape=[
            jax.ShapeDtypeStruct((n, _GI), jnp.int32),
            jax.ShapeDtypeStruct((n, _RP), jnp.float32),
        ],
    )(v, w_t, a_r)

    be = 1280
    ep2 = ep // 2
    rows2 = ep2 // _CH
    nbp = ep2 // (_NS * _CH)
    n0p = (nbp * 13) // 20 // 8 * 8
    de_t = dist_emb.T
    full = lambda shp: pl.BlockSpec(shp, lambda g: (0, 0))

    out_t = None
    for p in range(2):
        jp = lax.slice_in_dim(j_idx, p * rows2, (p + 1) * rows2, axis=0)
        ip = lax.slice_in_dim(i_idx, p * rows2, (p + 1) * rows2, axis=0)
        gv, gr = _sc_gather(vv, rt, jp, ip, ep2, nbp, n0p)

        goff = p * (ep2 // be)
        grid = (min(e, (p + 1) * ep2) - p * ep2) // be
        kwargs = {}
        args = []
        if p > 0:
            kwargs = dict(input_output_aliases={0: 0})
            args = [out_t]
        in_specs = ([full((8, _CH * 2))] if p > 0 else []) + [
            pl.BlockSpec((50, be), lambda g, goff=goff: (0, goff + g)),
            pl.BlockSpec((be, _GI), lambda g: (g, 0)),
            pl.BlockSpec((be, _RP), lambda g: (g, 0)),
            full((50, _NF)),
            full((1, _NF)),
            full((_NF, _DP)),
            full((1, _DP)),
            full((_DP, _NHP)),
            full((_DP, _NHP)),
            full((_NHP, _DP)),
        ]
        body = _tc2_body if p == 0 else (
            lambda carry_ref, *refs: _tc2_body(*refs))
        out_t = pl.pallas_call(
            body,
            grid=(grid,),
            in_specs=in_specs,
            out_specs=pl.BlockSpec((_D, be), lambda g, goff=goff: (0, goff + g)),
            out_shape=jax.ShapeDtypeStruct((_D, e), jnp.float32),
            **kwargs,
        )(*args, de_t, gv, gr, w0t, b0, w2t, b2, a_e, a_l, s_exp)

    return jnp.transpose(out_t.reshape(_NH, _NF, e), (2, 0, 1))


# speedup vs baseline: 1.6479x; 1.0345x over previous
import functools

import jax
import jax.numpy as jnp
from jax import lax
from jax.experimental import pallas as pl
from jax.experimental.pallas import tpu as pltpu
from jax.experimental.pallas import tpu_sc as plsc

_N = 10000
_E = 160000
_H = 128
_NH = 10
_NF = 32
_D = _NH * _NF
_NHP = 16

_DP = 384
_GP = 512
_GI = _GP // 2
_RP = 128

_NC = 2
_NS = 16
_NW = _NC * _NS
_CH = 64


def _ssp(x):
    return jnp.maximum(x, 0.0) + jnp.log1p(jnp.exp(-jnp.abs(x))) - jnp.log(2.0).astype(x.dtype)


def _pack_bf16(x):
    k = x.shape[1] // 2
    lo = jax.lax.bitcast_convert_type(x[:, :k].astype(jnp.bfloat16), jnp.uint16)
    hi = jax.lax.bitcast_convert_type(x[:, k:].astype(jnp.bfloat16), jnp.uint16)
    return (hi.astype(jnp.int32) << 16) | lo.astype(jnp.int32)


def _unpack_bf16(x):
    lo = jax.lax.bitcast_convert_type((x & 0xFFFF).astype(jnp.uint16),
                                      jnp.bfloat16)
    hi = jax.lax.bitcast_convert_type(
        jax.lax.shift_right_logical(x, 16).astype(jnp.uint16), jnp.bfloat16)
    return jnp.concatenate([lo.astype(jnp.float32), hi.astype(jnp.float32)],
                           axis=1)


def _tc1_body(v_ref, wt_ref, ar_ref, vv_ref, rt_ref):
    vv = jnp.dot(v_ref[...], wt_ref[...], preferred_element_type=jnp.float32)
    vv_ref[...] = _pack_bf16(vv)
    rt_ref[...] = jnp.dot(vv[:, :_DP], ar_ref[...],
                          preferred_element_type=jnp.float32)


def _tc2_body(det_ref, gv_ref, gr_ref, w0t_ref, b0_ref, w2t_ref, b2_ref,
              ae_ref, al_ref, sx_ref, out_ref):
    h = jax.lax.dot_general(det_ref[...], w0t_ref[...],
                            (((0,), (0,)), ((), ())),
                            preferred_element_type=jnp.float32)
    h = _ssp(h + b0_ref[...])
    w = jnp.dot(h, w2t_ref[...], preferred_element_type=jnp.float32) + b2_ref[...]
    gv = _unpack_bf16(gv_ref[...])[:, :_DP]
    ee = jnp.dot(w, ae_ref[...], preferred_element_type=jnp.float32)
    el = jnp.dot(gv, al_ref[...], preferred_element_type=jnp.float32)
    s = _ssp(el + ee + gr_ref[:, :_NHP])
    sx = jnp.dot(s, sx_ref[...], preferred_element_type=jnp.float32)
    out_ref[...] = jnp.transpose(gv * sx * w)[:_D, :]


def _sc_gather(vv, rt, idxj, idxi, ep, nb, n0):
    mesh = plsc.VectorSubcoreMesh(core_axis_name="c", subcore_axis_name="s")
    n1 = nb - n0
    nmax = max(n0, n1)
    nbuf = 4

    @functools.partial(
        pl.kernel,
        out_type=[
            jax.ShapeDtypeStruct((ep, _GI), jnp.int32),
            jax.ShapeDtypeStruct((ep, _RP), jnp.float32),
        ],
        mesh=mesh,
        scratch_types=[
            pltpu.VMEM((nmax, _CH), jnp.int32),
            pltpu.VMEM((nmax, _CH), jnp.int32),
            [pltpu.VMEM((_CH, _GI), jnp.int32)] * nbuf,
            [pltpu.VMEM((_CH, _RP), jnp.float32)] * nbuf,
            [pltpu.SemaphoreType.DMA] * nbuf,
            [pltpu.SemaphoreType.DMA] * nbuf,
            [pltpu.SemaphoreType.DMA] * nbuf,
            [pltpu.SemaphoreType.DMA] * nbuf,
        ],
    )
    def k(vv_hbm, rt_hbm, idxj_hbm, idxi_hbm, gv_hbm, gr_hbm,
          idxj_v, idxi_v, rv, rr, gs, go, hs, ho):
        cid = lax.axis_index("c")
        sid = lax.axis_index("s")
        crow = sid * nb + jnp.where(cid == 0, 0, n0)
        my_n = jnp.where(cid == 0, n0, n1)

        @pl.when(cid == 0)
        def _():
            pltpu.sync_copy(idxj_hbm.at[pl.ds(crow, n0)], idxj_v.at[pl.ds(0, n0)])
            pltpu.sync_copy(idxi_hbm.at[pl.ds(crow, n0)], idxi_v.at[pl.ds(0, n0)])

        @pl.when(cid == 1)
        def _():
            pltpu.sync_copy(idxj_hbm.at[pl.ds(crow, n1)], idxj_v.at[pl.ds(0, n1)])
            pltpu.sync_copy(idxi_hbm.at[pl.ds(crow, n1)], idxi_v.at[pl.ds(0, n1)])

        def g_rows(kk, b):
            return pltpu.make_async_copy(vv_hbm.at[idxj_v.at[kk]], rv[b], gs[b])

        def o_rows(kk, b):
            return pltpu.make_async_copy(
                rv[b], gv_hbm.at[pl.ds((crow + kk) * _CH, _CH)], go[b])

        def g_small(kk, b):
            return pltpu.make_async_copy(rt_hbm.at[idxi_v.at[kk]], rr[b], hs[b])

        def o_small(kk, b):
            return pltpu.make_async_copy(
                rr[b], gr_hbm.at[pl.ds((crow + kk) * _CH, _CH)], ho[b])

        for b in range(nbuf):
            g_rows(b, b).start()
            g_small(b, b).start()

        def body(g, carry):
            for b in range(nbuf):
                kk = nbuf * g + b
                g_rows(kk, b).wait()
                o_rows(kk, b).start()
                g_small(kk, b).wait()
                o_small(kk, b).start()

                @pl.when(kk + nbuf < my_n)
                def _():
                    o_rows(kk, b).wait()
                    g_rows(kk + nbuf, b).start()
                    o_small(kk, b).wait()
                    g_small(kk + nbuf, b).start()

            return carry

        lax.fori_loop(0, my_n // nbuf, body, 0)
        for b in range(nbuf):
            @pl.when(my_n - nbuf + b >= 0)
            def _(b=b):
                o_rows(my_n - nbuf + b, b).wait()
                o_small(my_n - nbuf + b, b).wait()

    return k(vv, rt, idxj, idxi)


def _blockdiag(a, nrow=_D, ncol=_NHP):
    out = jnp.zeros((nrow, ncol), jnp.float32)
    rows = jnp.arange(_D)
    cols = jnp.repeat(jnp.arange(_NH), _NF)
    return out.at[rows, cols].set(a.reshape(-1))


def kernel(v, dist, dist_emb, edge_index, lin_w, attn_l, attn_r, attn_edge,
           mlp_w0, mlp_b0, mlp_w2, mlp_b2):
    n, h = v.shape
    e = dist_emb.shape[0]

    a_l = _blockdiag(attn_l, _DP, _NHP)
    a_r = _blockdiag(attn_r, _DP, _RP)
    a_e = _blockdiag(attn_edge, _DP, _NHP)
    s_exp = jnp.zeros((_NHP, _DP), jnp.float32)
    s_exp = s_exp.at[jnp.repeat(jnp.arange(_NH), _NF), jnp.arange(_D)].set(1.0)
    w_t = jnp.pad(lin_w.T, ((0, 0), (0, _GP - _D)))
    w0t = mlp_w0.T
    w2t = jnp.pad(mlp_w2.T, ((0, 0), (0, _DP - _D)))
    b0 = mlp_b0.reshape(1, _NF)
    b2 = jnp.pad(mlp_b2.reshape(1, _D), ((0, 0), (0, _DP - _D)))

    nb = 16 * -(-e // (_NS * 16 * _CH))
    ep = nb * _NS * _CH
    n0 = (nb * 13) // 20 // 8 * 8
    j_idx = jnp.pad(edge_index[0], (0, ep - e)).reshape(ep // _CH, _CH)
    i_idx = jnp.pad(edge_index[1], (0, ep - e)).reshape(ep // _CH, _CH)

    vv, rt = pl.pallas_call(
        _tc1_body,
        out_shape=[
            jax.ShapeDtypeStruct((n, _GI), jnp.int32),
            jax.ShapeDtypeStruct((n, _RP), jnp.float32),
        ],
    )(v, w_t, a_r)

    nph = 4
    be = 1280
    ep2 = ep // nph
    rows2 = ep2 // _CH
    nbp = ep2 // (_NS * _CH)
    n0p = (nbp * 13) // 20 // 8 * 8
    de_t = dist_emb.T
    full = lambda shp: pl.BlockSpec(shp, lambda g: (0, 0))

    out_t = None
    for p in range(nph):
        jp = lax.slice_in_dim(j_idx, p * rows2, (p + 1) * rows2, axis=0)
        ip = lax.slice_in_dim(i_idx, p * rows2, (p + 1) * rows2, axis=0)
        gv, gr = _sc_gather(vv, rt, jp, ip, ep2, nbp, n0p)

        goff = p * (ep2 // be)
        grid = (min(e, (p + 1) * ep2) - p * ep2) // be
        kwargs = {}
        args = []
        if p > 0:
            kwargs = dict(input_output_aliases={0: 0})
            args = [out_t]
        in_specs = ([full((8, _CH * 2))] if p > 0 else []) + [
            pl.BlockSpec((50, be), lambda g, goff=goff: (0, goff + g)),
            pl.BlockSpec((be, _GI), lambda g: (g, 0)),
            pl.BlockSpec((be, _RP), lambda g: (g, 0)),
            full((50, _NF)),
            full((1, _NF)),
            full((_NF, _DP)),
            full((1, _DP)),
            full((_DP, _NHP)),
            full((_DP, _NHP)),
            full((_NHP, _DP)),
        ]
        body = _tc2_body if p == 0 else (
            lambda carry_ref, *refs: _tc2_body(*refs))
        out_t = pl.pallas_call(
            body,
            grid=(grid,),
            in_specs=in_specs,
            out_specs=pl.BlockSpec((_D, be), lambda g, goff=goff: (0, goff + g)),
            out_shape=jax.ShapeDtypeStruct((_D, e), jnp.float32),
            **kwargs,
        )(*args, de_t, gv, gr, w0t, b0, w2t, b2, a_e, a_l, s_exp)

    return jnp.transpose(out_t.reshape(_NH, _NF, e), (2, 0, 1))


# speedup vs baseline: 1.6655x; 1.0107x over previous
import functools

import jax
import jax.numpy as jnp
from jax import lax
from jax.experimental import pallas as pl
from jax.experimental.pallas import tpu as pltpu
from jax.experimental.pallas import tpu_sc as plsc

_N = 10000
_E = 160000
_H = 128
_NH = 10
_NF = 32
_D = _NH * _NF
_NHP = 16

_DP = 384
_GP = 512
_GI = _GP // 2
_RP = 128

_NC = 2
_NS = 16
_NW = _NC * _NS
_CH = 64


def _ssp(x):
    return jnp.maximum(x, 0.0) + jnp.log1p(jnp.exp(-jnp.abs(x))) - jnp.log(2.0).astype(x.dtype)


def _pack_bf16(x):
    k = x.shape[1] // 2
    lo = jax.lax.bitcast_convert_type(x[:, :k].astype(jnp.bfloat16), jnp.uint16)
    hi = jax.lax.bitcast_convert_type(x[:, k:].astype(jnp.bfloat16), jnp.uint16)
    return (hi.astype(jnp.int32) << 16) | lo.astype(jnp.int32)


def _unpack_bf16(x):
    lo = jax.lax.bitcast_convert_type((x & 0xFFFF).astype(jnp.uint16),
                                      jnp.bfloat16)
    hi = jax.lax.bitcast_convert_type(
        jax.lax.shift_right_logical(x, 16).astype(jnp.uint16), jnp.bfloat16)
    return jnp.concatenate([lo.astype(jnp.float32), hi.astype(jnp.float32)],
                           axis=1)


def _tc1_body(v_ref, wt_ref, ar_ref, vv_ref, rt_ref):
    vv = jnp.dot(v_ref[...], wt_ref[...], preferred_element_type=jnp.float32)
    vv_ref[...] = _pack_bf16(vv)
    rt_ref[...] = jnp.dot(vv[:, :_DP], ar_ref[...],
                          preferred_element_type=jnp.float32)


def _tc2_body(det_ref, gv_ref, gr_ref, w0t_ref, b0_ref, w2t_ref, b2_ref,
              ae_ref, al_ref, sx_ref, out_ref):
    h = jax.lax.dot_general(det_ref[...], w0t_ref[...],
                            (((0,), (0,)), ((), ())),
                            preferred_element_type=jnp.float32)
    h = _ssp(h + b0_ref[...])
    w = jnp.dot(h, w2t_ref[...], preferred_element_type=jnp.float32) + b2_ref[...]
    gv = _unpack_bf16(gv_ref[...])[:, :_DP]
    ee = jnp.dot(w, ae_ref[...], preferred_element_type=jnp.float32)
    el = jnp.dot(gv, al_ref[...], preferred_element_type=jnp.float32)
    s = _ssp(el + ee + gr_ref[:, :_NHP])
    sx = jnp.dot(s, sx_ref[...], preferred_element_type=jnp.float32)
    out_ref[...] = jnp.transpose(gv * sx * w)[:_D, :]


def _sc_gather(vv, rt, idxj, idxi, ep, nb, n0):
    mesh = plsc.VectorSubcoreMesh(core_axis_name="c", subcore_axis_name="s")
    n1 = nb - n0
    nmax = max(n0, n1)
    nbuf = 4

    @functools.partial(
        pl.kernel,
        out_type=[
            jax.ShapeDtypeStruct((ep, _GI), jnp.int32),
            jax.ShapeDtypeStruct((ep, _RP), jnp.float32),
        ],
        mesh=mesh,
        scratch_types=[
            pltpu.VMEM((nmax, _CH), jnp.int32),
            pltpu.VMEM((nmax, _CH), jnp.int32),
            [pltpu.VMEM((_CH, _GI), jnp.int32)] * nbuf,
            [pltpu.VMEM((_CH, _RP), jnp.float32)] * nbuf,
            [pltpu.SemaphoreType.DMA] * nbuf,
            [pltpu.SemaphoreType.DMA] * nbuf,
            [pltpu.SemaphoreType.DMA] * nbuf,
            [pltpu.SemaphoreType.DMA] * nbuf,
        ],
    )
    def k(vv_hbm, rt_hbm, idxj_hbm, idxi_hbm, gv_hbm, gr_hbm,
          idxj_v, idxi_v, rv, rr, gs, go, hs, ho):
        cid = lax.axis_index("c")
        sid = lax.axis_index("s")
        crow = sid * nb + jnp.where(cid == 0, 0, n0)
        my_n = jnp.where(cid == 0, n0, n1)

        @pl.when(cid == 0)
        def _():
            pltpu.sync_copy(idxj_hbm.at[pl.ds(crow, n0)], idxj_v.at[pl.ds(0, n0)])
            pltpu.sync_copy(idxi_hbm.at[pl.ds(crow, n0)], idxi_v.at[pl.ds(0, n0)])

        @pl.when(cid == 1)
        def _():
            pltpu.sync_copy(idxj_hbm.at[pl.ds(crow, n1)], idxj_v.at[pl.ds(0, n1)])
            pltpu.sync_copy(idxi_hbm.at[pl.ds(crow, n1)], idxi_v.at[pl.ds(0, n1)])

        def g_rows(kk, b):
            return pltpu.make_async_copy(vv_hbm.at[idxj_v.at[kk]], rv[b], gs[b])

        def o_rows(kk, b):
            return pltpu.make_async_copy(
                rv[b], gv_hbm.at[pl.ds((crow + kk) * _CH, _CH)], go[b])

        def g_small(kk, b):
            return pltpu.make_async_copy(rt_hbm.at[idxi_v.at[kk]], rr[b], hs[b])

        def o_small(kk, b):
            return pltpu.make_async_copy(
                rr[b], gr_hbm.at[pl.ds((crow + kk) * _CH, _CH)], ho[b])

        for b in range(nbuf):
            g_rows(b, b).start()
            g_small(b, b).start()

        def body(g, carry):
            for b in range(nbuf):
                kk = nbuf * g + b
                g_rows(kk, b).wait()
                o_rows(kk, b).start()
                g_small(kk, b).wait()
                o_small(kk, b).start()

                @pl.when(kk + nbuf < my_n)
                def _():
                    o_rows(kk, b).wait()
                    g_rows(kk + nbuf, b).start()
                    o_small(kk, b).wait()
                    g_small(kk + nbuf, b).start()

            return carry

        lax.fori_loop(0, my_n // nbuf, body, 0)
        for b in range(nbuf):
            @pl.when(my_n - nbuf + b >= 0)
            def _(b=b):
                o_rows(my_n - nbuf + b, b).wait()
                o_small(my_n - nbuf + b, b).wait()

    return k(vv, rt, idxj, idxi)


def _blockdiag(a, nrow=_D, ncol=_NHP):
    out = jnp.zeros((nrow, ncol), jnp.float32)
    rows = jnp.arange(_D)
    cols = jnp.repeat(jnp.arange(_NH), _NF)
    return out.at[rows, cols].set(a.reshape(-1))


def kernel(v, dist, dist_emb, edge_index, lin_w, attn_l, attn_r, attn_edge,
           mlp_w0, mlp_b0, mlp_w2, mlp_b2):
    n, h = v.shape
    e = dist_emb.shape[0]

    a_l = _blockdiag(attn_l, _DP, _NHP)
    a_r = _blockdiag(attn_r, _DP, _RP)
    a_e = _blockdiag(attn_edge, _DP, _NHP)
    s_exp = jnp.zeros((_NHP, _DP), jnp.float32)
    s_exp = s_exp.at[jnp.repeat(jnp.arange(_NH), _NF), jnp.arange(_D)].set(1.0)
    w_t = jnp.pad(lin_w.T, ((0, 0), (0, _GP - _D)))
    w0t = mlp_w0.T
    w2t = jnp.pad(mlp_w2.T, ((0, 0), (0, _DP - _D)))
    b0 = mlp_b0.reshape(1, _NF)
    b2 = jnp.pad(mlp_b2.reshape(1, _D), ((0, 0), (0, _DP - _D)))

    nb = 16 * -(-e // (_NS * 16 * _CH))
    ep = nb * _NS * _CH
    n0 = (nb * 13) // 20 // 8 * 8
    j_idx = jnp.pad(edge_index[0], (0, ep - e)).reshape(ep // _CH, _CH)
    i_idx = jnp.pad(edge_index[1], (0, ep - e)).reshape(ep // _CH, _CH)

    vv, rt = pl.pallas_call(
        _tc1_body,
        out_shape=[
            jax.ShapeDtypeStruct((n, _GI), jnp.int32),
            jax.ShapeDtypeStruct((n, _RP), jnp.float32),
        ],
    )(v, w_t, a_r)

    nph = 4
    ep2 = ep // nph
    rows2 = ep2 // _CH
    nbp = ep2 // (_NS * _CH)
    n0p = (nbp * 13) // 20 // 8 * 8
    de_t = dist_emb.T
    full = lambda shp: pl.BlockSpec(shp, lambda g: (0, 0))

    out_t = None
    for p in range(nph):
        jp = lax.slice_in_dim(j_idx, p * rows2, (p + 1) * rows2, axis=0)
        ip = lax.slice_in_dim(i_idx, p * rows2, (p + 1) * rows2, axis=0)
        gv, gr = _sc_gather(vv, rt, jp, ip, ep2, nbp, n0p)

        sz = min(e, (p + 1) * ep2) - p * ep2
        be = next(b for b in (4096, 2048, 1280, 640, 256, 128) if sz % b == 0
                  and (p * ep2) % b == 0)
        goff = p * ep2 // be
        grid = sz // be
        kwargs = {}
        args = []
        if p > 0:
            kwargs = dict(input_output_aliases={0: 0})
            args = [out_t]
        in_specs = ([full((8, _CH * 2))] if p > 0 else []) + [
            pl.BlockSpec((50, be), lambda g, goff=goff: (0, goff + g)),
            pl.BlockSpec((be, _GI), lambda g: (g, 0)),
            pl.BlockSpec((be, _RP), lambda g: (g, 0)),
            full((50, _NF)),
            full((1, _NF)),
            full((_NF, _DP)),
            full((1, _DP)),
            full((_DP, _NHP)),
            full((_DP, _NHP)),
            full((_NHP, _DP)),
        ]
        body = _tc2_body if p == 0 else (
            lambda carry_ref, *refs: _tc2_body(*refs))
        out_t = pl.pallas_call(
            body,
            grid=(grid,),
            in_specs=in_specs,
            out_specs=pl.BlockSpec((_D, be), lambda g, goff=goff: (0, goff + g)),
            out_shape=jax.ShapeDtypeStruct((_D, e), jnp.float32),
            **kwargs,
        )(*args, de_t, gv, gr, w0t, b0, w2t, b2, a_e, a_l, s_exp)

    return jnp.transpose(out_t.reshape(_NH, _NF, e), (2, 0, 1))


# speedup vs baseline: 1.7294x; 1.0384x over previous
import functools

import jax
import jax.numpy as jnp
from jax import lax
from jax.experimental import pallas as pl
from jax.experimental.pallas import tpu as pltpu
from jax.experimental.pallas import tpu_sc as plsc

_N = 10000
_E = 160000
_H = 128
_NH = 10
_NF = 32
_D = _NH * _NF
_NHP = 16

_DP = 384
_GP = 512
_GI = _GP // 2
_RP = 128

_NC = 2
_NS = 16
_NW = _NC * _NS
_CH = 64


def _ssp(x):
    return jnp.maximum(x, 0.0) + jnp.log1p(jnp.exp(-jnp.abs(x))) - jnp.log(2.0).astype(x.dtype)


def _pack_bf16(x):
    k = x.shape[1] // 2
    lo = jax.lax.bitcast_convert_type(x[:, :k].astype(jnp.bfloat16), jnp.uint16)
    hi = jax.lax.bitcast_convert_type(x[:, k:].astype(jnp.bfloat16), jnp.uint16)
    return (hi.astype(jnp.int32) << 16) | lo.astype(jnp.int32)


def _unpack_bf16(x):
    lo = jax.lax.bitcast_convert_type((x & 0xFFFF).astype(jnp.uint16),
                                      jnp.bfloat16)
    hi = jax.lax.bitcast_convert_type(
        jax.lax.shift_right_logical(x, 16).astype(jnp.uint16), jnp.bfloat16)
    return jnp.concatenate([lo.astype(jnp.float32), hi.astype(jnp.float32)],
                           axis=1)


def _tc1_body(v_ref, wt_ref, ar_ref, vv_ref, rt_ref):
    vv = jnp.dot(v_ref[...], wt_ref[...], preferred_element_type=jnp.float32)
    vv_ref[...] = _pack_bf16(vv)
    rt_ref[...] = jnp.dot(vv[:, :_DP], ar_ref[...],
                          preferred_element_type=jnp.float32)


def _tc2_body(det_ref, gv_ref, gr_ref, w0t_ref, b0_ref, w2t_ref, b2_ref,
              ae_ref, al_ref, sx_ref, out_ref):
    h = jax.lax.dot_general(det_ref[...], w0t_ref[...],
                            (((0,), (0,)), ((), ())),
                            preferred_element_type=jnp.float32)
    h = _ssp(h + b0_ref[...])
    w = jnp.dot(h, w2t_ref[...], preferred_element_type=jnp.float32) + b2_ref[...]
    gv = _unpack_bf16(gv_ref[...])[:, :_DP]
    ee = jnp.dot(w, ae_ref[...], preferred_element_type=jnp.float32)
    el = jnp.dot(gv, al_ref[...], preferred_element_type=jnp.float32)
    s = _ssp(el + ee + gr_ref[:, :_NHP])
    sx = jnp.dot(s, sx_ref[...], preferred_element_type=jnp.float32)
    out_ref[...] = jnp.transpose(gv * sx * w)[:_D, :]


def _sc_gather(vv, rt, idxj, idxi, ep, nb, n0):
    mesh = plsc.VectorSubcoreMesh(core_axis_name="c", subcore_axis_name="s")
    n1 = nb - n0
    nmax = max(n0, n1)
    nbuf = 4

    @functools.partial(
        pl.kernel,
        out_type=[
            jax.ShapeDtypeStruct((ep, _GI), jnp.int32),
            jax.ShapeDtypeStruct((ep, _RP), jnp.float32),
        ],
        mesh=mesh,
        scratch_types=[
            pltpu.VMEM((nmax, _CH), jnp.int32),
            pltpu.VMEM((nmax, _CH), jnp.int32),
            [pltpu.VMEM((_CH, _GI), jnp.int32)] * nbuf,
            [pltpu.VMEM((_CH, _RP), jnp.float32)] * nbuf,
            [pltpu.SemaphoreType.DMA] * nbuf,
            [pltpu.SemaphoreType.DMA] * nbuf,
            [pltpu.SemaphoreType.DMA] * nbuf,
            [pltpu.SemaphoreType.DMA] * nbuf,
        ],
    )
    def k(vv_hbm, rt_hbm, idxj_hbm, idxi_hbm, gv_hbm, gr_hbm,
          idxj_v, idxi_v, rv, rr, gs, go, hs, ho):
        cid = lax.axis_index("c")
        sid = lax.axis_index("s")
        crow = sid * nb + jnp.where(cid == 0, 0, n0)
        my_n = jnp.where(cid == 0, n0, n1)

        @pl.when(cid == 0)
        def _():
            pltpu.sync_copy(idxj_hbm.at[pl.ds(crow, n0)], idxj_v.at[pl.ds(0, n0)])
            pltpu.sync_copy(idxi_hbm.at[pl.ds(crow, n0)], idxi_v.at[pl.ds(0, n0)])

        @pl.when(cid == 1)
        def _():
            pltpu.sync_copy(idxj_hbm.at[pl.ds(crow, n1)], idxj_v.at[pl.ds(0, n1)])
            pltpu.sync_copy(idxi_hbm.at[pl.ds(crow, n1)], idxi_v.at[pl.ds(0, n1)])

        def g_rows(kk, b):
            return pltpu.make_async_copy(vv_hbm.at[idxj_v.at[kk]], rv[b], gs[b])

        def o_rows(kk, b):
            return pltpu.make_async_copy(
                rv[b], gv_hbm.at[pl.ds((crow + kk) * _CH, _CH)], go[b])

        def g_small(kk, b):
            return pltpu.make_async_copy(rt_hbm.at[idxi_v.at[kk]], rr[b], hs[b])

        def o_small(kk, b):
            return pltpu.make_async_copy(
                rr[b], gr_hbm.at[pl.ds((crow + kk) * _CH, _CH)], ho[b])

        for b in range(nbuf):
            g_rows(b, b).start()
            g_small(b, b).start()

        def body(g, carry):
            for b in range(nbuf):
                kk = nbuf * g + b
                g_rows(kk, b).wait()
                o_rows(kk, b).start()
                g_small(kk, b).wait()
                o_small(kk, b).start()

                @pl.when(kk + nbuf < my_n)
                def _():
                    o_rows(kk, b).wait()
                    g_rows(kk + nbuf, b).start()
                    o_small(kk, b).wait()
                    g_small(kk + nbuf, b).start()

            return carry

        lax.fori_loop(0, my_n // nbuf, body, 0)
        for b in range(nbuf):
            @pl.when(my_n - nbuf + b >= 0)
            def _(b=b):
                o_rows(my_n - nbuf + b, b).wait()
                o_small(my_n - nbuf + b, b).wait()

    return k(vv, rt, idxj, idxi)


def _blockdiag(a, nrow=_D, ncol=_NHP):
    rows = jnp.arange(nrow)
    vals = jnp.pad(a.reshape(-1), (0, nrow - _D))
    mask = (jnp.arange(ncol)[None, :] == (rows // _NF)[:, None]) & (
        rows < _D)[:, None]
    return jnp.where(mask, vals[:, None], 0.0)


def kernel(v, dist, dist_emb, edge_index, lin_w, attn_l, attn_r, attn_edge,
           mlp_w0, mlp_b0, mlp_w2, mlp_b2):
    n, h = v.shape
    e = dist_emb.shape[0]

    a_l = _blockdiag(attn_l, _DP, _NHP)
    a_r = _blockdiag(attn_r, _DP, _RP)
    a_e = _blockdiag(attn_edge, _DP, _NHP)
    cols = jnp.arange(_DP)
    s_exp = ((jnp.arange(_NHP)[:, None] == cols[None, :] // _NF) & (
        cols < _D)[None, :]).astype(jnp.float32)
    w_t = jnp.pad(lin_w.T, ((0, 0), (0, _GP - _D)))
    w0t = mlp_w0.T
    w2t = jnp.pad(mlp_w2.T, ((0, 0), (0, _DP - _D)))
    b0 = mlp_b0.reshape(1, _NF)
    b2 = jnp.pad(mlp_b2.reshape(1, _D), ((0, 0), (0, _DP - _D)))

    nb = 16 * -(-e // (_NS * 16 * _CH))
    ep = nb * _NS * _CH
    n0 = (nb * 13) // 20 // 8 * 8
    j_idx = jnp.pad(edge_index[0], (0, ep - e)).reshape(ep // _CH, _CH)
    i_idx = jnp.pad(edge_index[1], (0, ep - e)).reshape(ep // _CH, _CH)

    vv, rt = pl.pallas_call(
        _tc1_body,
        out_shape=[
            jax.ShapeDtypeStruct((n, _GI), jnp.int32),
            jax.ShapeDtypeStruct((n, _RP), jnp.float32),
        ],
    )(v, w_t, a_r)

    nph = 4
    ep2 = ep // nph
    rows2 = ep2 // _CH
    nbp = ep2 // (_NS * _CH)
    n0p = (nbp * 13) // 20 // 8 * 8
    de_t = dist_emb.T
    full = lambda shp: pl.BlockSpec(shp, lambda g: (0, 0))

    out_t = None
    for p in range(nph):
        jp = lax.slice_in_dim(j_idx, p * rows2, (p + 1) * rows2, axis=0)
        ip = lax.slice_in_dim(i_idx, p * rows2, (p + 1) * rows2, axis=0)
        gv, gr = _sc_gather(vv, rt, jp, ip, ep2, nbp, n0p)

        sz = min(e, (p + 1) * ep2) - p * ep2
        be = next(b for b in (4096, 2048, 1280, 640, 256, 128) if sz % b == 0
                  and (p * ep2) % b == 0)
        goff = p * ep2 // be
        grid = sz // be
        kwargs = {}
        args = []
        if p > 0:
            kwargs = dict(input_output_aliases={0: 0})
            args = [out_t]
        in_specs = ([full((8, _CH * 2))] if p > 0 else []) + [
            pl.BlockSpec((50, be), lambda g, goff=goff: (0, goff + g)),
            pl.BlockSpec((be, _GI), lambda g: (g, 0)),
            pl.BlockSpec((be, _RP), lambda g: (g, 0)),
            full((50, _NF)),
            full((1, _NF)),
            full((_NF, _DP)),
            full((1, _DP)),
            full((_DP, _NHP)),
            full((_DP, _NHP)),
            full((_NHP, _DP)),
        ]
        body = _tc2_body if p == 0 else (
            lambda carry_ref, *refs: _tc2_body(*refs))
        out_t = pl.pallas_call(
            body,
            grid=(grid,),
            in_specs=in_specs,
            out_specs=pl.BlockSpec((_D, be), lambda g, goff=goff: (0, goff + g)),
            out_shape=jax.ShapeDtypeStruct((_D, e), jnp.float32),
            **kwargs,
        )(*args, de_t, gv, gr, w0t, b0, w2t, b2, a_e, a_l, s_exp)

    return jnp.transpose(out_t.reshape(_NH, _NF, e), (2, 0, 1))
